# baseline (speedup 1.0000x reference)
import functools

import jax
import jax.numpy as jnp
from jax import lax
from jax.experimental import pallas as pl
from jax.experimental.pallas import tpu as pltpu
from jax.experimental.pallas import tpu_sc as plsc

HID = 512
CHUNK = 128
NCHUNK = HID // CHUNK
NSUB = 16
NCORE = 2
EB = 128
NBATCH = 10
E_PAD = NSUB * NBATCH * EB
BIG = 10240
SMALL = 1024
BM = 1000


def _make_seg_sum(spec, n_slots, n_sidx, n_didx):
    n_et = len(spec)
    mesh = plsc.VectorSubcoreMesh(core_axis_name="c", subcore_axis_name="s")
    out_type = [jax.ShapeDtypeStruct((NCHUNK, np_, CHUNK), jnp.float32)
                for _, _, _, np_, _ in spec]
    scratch = [
        pltpu.VMEM((EB,), jnp.int32),
        pltpu.VMEM((EB,), jnp.int32),
        pltpu.VMEM((EB, CHUNK), jnp.float32),
        pltpu.VMEM_SHARED((BIG, CHUNK), jnp.float32),
        pltpu.SemaphoreType.DMA,
    ]

    @functools.partial(pl.kernel, mesh=mesh, out_type=out_type,
                       scratch_types=scratch)
    def seg_sum(*refs):
        srcs = refs[:n_slots]
        sidx = refs[n_slots:n_slots + n_sidx]
        didx = refs[n_slots + n_sidx:n_slots + n_sidx + n_didx]
        zeros_h = refs[n_slots + n_sidx + n_didx]
        outs = refs[n_slots + n_sidx + n_didx + 1:
                    n_slots + n_sidx + n_didx + 1 + n_et]
        iv_s, iv_d, rows, acc, sem = refs[n_slots + n_sidx + n_didx + 1 + n_et:]
        c = lax.axis_index("c")
        s = lax.axis_index("s")

        def one_pass(g, e, slot, si, di, n_pad):
            stripe = n_pad // NSUB
            r0 = s * stripe
            pltpu.sync_copy(zeros_h.at[pl.ds(0, stripe)],
                            acc.at[pl.ds(r0, stripe)])
            plsc.subcore_barrier()

            def batch(j, carry):
                base = (s * NBATCH + j) * EB
                pltpu.sync_copy(sidx[si].at[pl.ds(base, EB)], iv_s)
                pltpu.sync_copy(didx[di].at[pl.ds(base, EB)], iv_d)
                pltpu.async_copy(srcs[slot].at[g].at[iv_s], rows, sem).wait()
                pltpu.sync_copy(rows, acc.at[iv_d], add=True)
                return carry

            lax.fori_loop(0, NBATCH, batch, 0)
            plsc.subcore_barrier()
            pltpu.sync_copy(acc.at[pl.ds(r0, stripe)],
                            outs[e].at[g].at[pl.ds(r0, stripe)])
            plsc.subcore_barrier()

        for q in range(NCHUNK // NCORE):
            gq = c * (NCHUNK // NCORE) + q
            for e, (slot, si, di, n_pad, owner) in enumerate(spec):
                if owner is None:
                    one_pass(gq, e, slot, si, di, n_pad)
                elif q == 0:
                    @pl.when(c == owner)
                    def _(e=e, slot=slot, si=si, di=di, n_pad=n_pad):
                        one_pass(0, e, slot, si, di, n_pad)

    return seg_sum


def _sage_matmul(a_parts, cnts, x_root, w, bias, *, m, relu, out_chunked,
                 root_chunked, bm=BM):
    nseg = len(a_parts)
    nk = nseg + 1
    grid = (m // bm, nk)

    def body(*refs):
        a_refs = refs[:nseg]
        c_refs = refs[nseg:2 * nseg]
        x_ref = refs[2 * nseg]
        w_ref = refs[2 * nseg + 1]
        b_ref = refs[2 * nseg + 2]
        o_ref = refs[2 * nseg + 3]
        acc = refs[2 * nseg + 4]
        k = pl.program_id(1)

        @pl.when(k == 0)
        def _():
            acc[...] = jnp.zeros_like(acc)

        for e in range(nseg):
            @pl.when(k == e)
            def _(e=e):
                scale = 1.0 / jnp.maximum(c_refs[e][0][:, 0:1], 1.0)
                for g in range(NCHUNK):
                    acc[...] += jnp.dot(
                        a_refs[e][g] * scale,
                        w_ref[g * CHUNK:(g + 1) * CHUNK, :],
                        preferred_element_type=jnp.float32)

        @pl.when(k == nseg)
        def _():
            if root_chunked:
                for g in range(NCHUNK):
                    acc[...] += jnp.dot(
                        x_ref[g], w_ref[g * CHUNK:(g + 1) * CHUNK, :],
                        preferred_element_type=jnp.float32)
            else:
                acc[...] += jnp.dot(x_ref[...], w_ref[...],
                                    preferred_element_type=jnp.float32)
            res = acc[...] + b_ref[0:1, :]
            if relu:
                res = jnp.maximum(res, 0.0)
            if out_chunked:
                for g in range(NCHUNK):
                    o_ref[g] = res[:, g * CHUNK:(g + 1) * CHUNK]
            else:
                o_ref[...] = res

    in_specs = (
        [pl.BlockSpec((NCHUNK, bm, CHUNK), lambda i, k: (0, i, 0))
         for _ in range(nseg)]
        + [pl.BlockSpec((1, bm, CHUNK), lambda i, k: (0, i, 0))
           for _ in range(nseg)]
        + [pl.BlockSpec((NCHUNK, bm, CHUNK), lambda i, k: (0, i, 0))
           if root_chunked else pl.BlockSpec((bm, HID), lambda i, k: (i, 0))]
        + [pl.BlockSpec((HID, HID), lambda i, k: (k, 0)),
           pl.BlockSpec((8, HID), lambda i, k: (0, 0))]
    )
    if out_chunked:
        out_spec = pl.BlockSpec((NCHUNK, bm, CHUNK), lambda i, k: (0, i, 0))
        out_shape = jax.ShapeDtypeStruct((NCHUNK, m, CHUNK), jnp.float32)
    else:
        out_spec = pl.BlockSpec((bm, HID), lambda i, k: (i, 0))
        out_shape = jax.ShapeDtypeStruct((m, HID), jnp.float32)

    return pl.pallas_call(
        body, grid=grid, in_specs=in_specs, out_specs=out_spec,
        out_shape=out_shape,
        scratch_shapes=[pltpu.VMEM((bm, HID), jnp.float32)],
        compiler_params=pltpu.CompilerParams(
            dimension_semantics=("parallel", "arbitrary")),
    )(*a_parts, *cnts, x_root, w, bias)


def _final_linear(x, w, bias, *, m, n_out, bm=BM):
    def body(x_ref, w_ref, b_ref, o_ref):
        o_ref[...] = jnp.dot(x_ref[...], w_ref[...],
                             preferred_element_type=jnp.float32) + b_ref[0:1, :]

    return pl.pallas_call(
        body, grid=(m // bm,),
        in_specs=[pl.BlockSpec((bm, HID), lambda i: (i, 0)),
                  pl.BlockSpec((HID, n_out), lambda i: (0, 0)),
                  pl.BlockSpec((8, n_out), lambda i: (0, 0))],
        out_specs=pl.BlockSpec((bm, n_out), lambda i: (i, 0)),
        out_shape=jax.ShapeDtypeStruct((m, n_out), jnp.float32),
        compiler_params=pltpu.CompilerParams(
            dimension_semantics=("parallel",)),
    )(x, w, bias)


def _chunked(x):
    n = x.shape[0]
    return x.reshape(n, NCHUNK, CHUNK).transpose(1, 0, 2)


def kernel(x_paper, x_author, x_institution, x_field_of_study, ei_cites,
           ei_writes, ei_rev_writes, ei_affiliated, ei_rev_affiliated,
           ei_has_topic, ei_rev_has_topic, Wl, bl, Wr, lin_W, lin_b):
    f32 = jnp.float32
    n_paper = x_paper.shape[0]
    n_author = x_author.shape[0]
    n_fos = x_field_of_study.shape[0]
    e = ei_cites.shape[1]
    pad = E_PAD - e

    def prep(ei, n_dst):
        src = jnp.concatenate([ei[0], jnp.zeros((pad,), jnp.int32)])
        dst = jnp.concatenate([ei[1], jnp.full((pad,), n_dst, jnp.int32)])
        return src, dst

    s_ci, d_ci = prep(ei_cites, n_paper)
    s_wr, d_wr = prep(ei_writes, n_paper)
    s_rw, d_rw = prep(ei_rev_writes, n_author)
    s_ra, d_ra = prep(ei_rev_affiliated, n_author)
    s_ht, d_ht = prep(ei_has_topic, n_fos)
    s_rh, d_rh = prep(ei_rev_has_topic, n_paper)
    z_idx = jnp.zeros((E_PAD,), jnp.int32)

    zeros_big = jnp.zeros((BIG // NSUB, CHUNK), f32)
    ones_src = jnp.ones((1, 8, CHUNK), f32)

    xc_p = _chunked(x_paper)
    xc_a = _chunked(x_author)
    xc_i = _chunked(x_institution)
    xc_f = _chunked(x_field_of_study)
    spec1 = (
        (0, 0, 0, BIG, None),
        (1, 1, 1, BIG, None),
        (0, 2, 2, BIG, None),
        (2, 3, 3, BIG, None),
        (0, 4, 4, SMALL, None),
        (3, 5, 5, BIG, None),
        (4, 6, 0, BIG, 0),
        (4, 6, 1, BIG, 1),
        (4, 6, 2, BIG, 0),
        (4, 6, 3, BIG, 1),
        (4, 6, 4, SMALL, 0),
        (4, 6, 5, BIG, 1),
    )
    (agg_ci, agg_wr, agg_rw, agg_ra, agg_ht, agg_rh,
     cnt_ci, cnt_wr, cnt_rw, cnt_ra, cnt_ht, cnt_rh) = _make_seg_sum(
        spec1, 5, 7, 6)(xc_p, xc_a, xc_i, xc_f, ones_src,
                        s_ci, s_wr, s_rw, s_ra, s_ht, s_rh, z_idx,
                        d_ci, d_wr, d_rw, d_ra, d_ht, d_rh, zeros_big)

    def w_cat(layer, js):
        parts = [Wl[layer, j].T for j in js]
        parts.append(sum(Wr[layer, j] for j in js).T)
        return jnp.concatenate(parts, axis=0)

    def b_sum(layer, js):
        b = sum(bl[layer, j] for j in js)
        return jnp.broadcast_to(b[None, :], (8, HID))

    x1_p = _sage_matmul([agg_ci, agg_wr, agg_rh], [cnt_ci, cnt_wr, cnt_rh],
                        x_paper, w_cat(0, (0, 1, 6)), b_sum(0, (0, 1, 6)),
                        m=n_paper, relu=True, out_chunked=True,
                        root_chunked=False)
    x1_a = _sage_matmul([agg_rw, agg_ra], [cnt_rw, cnt_ra],
                        x_author, w_cat(0, (2, 4)), b_sum(0, (2, 4)),
                        m=n_author, relu=True, out_chunked=True,
                        root_chunked=False)
    x1_f = _sage_matmul([agg_ht], [cnt_ht],
                        x_field_of_study, w_cat(0, (5,)), b_sum(0, (5,)),
                        m=n_fos, relu=True, out_chunked=True,
                        root_chunked=False)

    spec2 = ((0, 0, 0, BIG, None), (1, 1, 1, BIG, None),
             (2, 2, 2, BIG, None))
    agg2_ci, agg2_wr, agg2_rh = _make_seg_sum(spec2, 3, 3, 3)(
        x1_p, x1_a, x1_f, s_ci, s_wr, s_rh, d_ci, d_wr, d_rh, zeros_big)

    x2_p = _sage_matmul([agg2_ci, agg2_wr, agg2_rh],
                        [cnt_ci, cnt_wr, cnt_rh],
                        x1_p, w_cat(1, (0, 1, 6)), b_sum(1, (0, 1, 6)),
                        m=n_paper, relu=True, out_chunked=False,
                        root_chunked=True)

    n_out = lin_W.shape[0]
    lin_bias = jnp.broadcast_to(lin_b[None, :], (8, n_out))
    return _final_linear(x2_p, lin_W.T, lin_bias, m=n_paper, n_out=n_out)


# speedup vs baseline: 3.6637x; 3.6637x over previous
import functools

import jax
import jax.numpy as jnp
from jax import lax
from jax.experimental import pallas as pl
from jax.experimental.pallas import tpu as pltpu
from jax.experimental.pallas import tpu_sc as plsc

HID = 512
CHUNK = 128
NCHUNK = HID // CHUNK
NSUB = 16
NCORE = 2
EB = 128
NBATCH = 10
E_PAD = NSUB * NBATCH * EB
BIG = 10240
SMALL = 1024
BM = 1000


def _make_seg_sum(spec, n_slots, n_sidx, n_didx):
    n_et = len(spec)
    mesh = plsc.VectorSubcoreMesh(core_axis_name="c", subcore_axis_name="s")
    out_type = [jax.ShapeDtypeStruct((NCHUNK, np_, CHUNK), jnp.float32)
                for _, _, _, np_, _ in spec]
    scratch = [
        pltpu.VMEM((EB,), jnp.int32),
        pltpu.VMEM((EB,), jnp.int32),
        pltpu.VMEM((EB, CHUNK), jnp.float32),
        pltpu.VMEM_SHARED((BIG, CHUNK), jnp.float32),
        pltpu.SemaphoreType.DMA,
    ]

    @functools.partial(pl.kernel, mesh=mesh, out_type=out_type,
                       scratch_types=scratch)
    def seg_sum(*refs):
        srcs = refs[:n_slots]
        sidx = refs[n_slots:n_slots + n_sidx]
        didx = refs[n_slots + n_sidx:n_slots + n_sidx + n_didx]
        zeros_h = refs[n_slots + n_sidx + n_didx]
        outs = refs[n_slots + n_sidx + n_didx + 1:
                    n_slots + n_sidx + n_didx + 1 + n_et]
        iv_s, iv_d, rows, acc, sem = refs[n_slots + n_sidx + n_didx + 1 + n_et:]
        c = lax.axis_index("c")
        s = lax.axis_index("s")

        def one_pass(g, e, slot, si, di, n_pad):
            stripe = n_pad // NSUB
            r0 = s * stripe
            pltpu.sync_copy(zeros_h.at[pl.ds(0, stripe)],
                            acc.at[pl.ds(r0, stripe)])
            plsc.subcore_barrier()

            def batch(j, carry):
                base = (s * NBATCH + j) * EB
                pltpu.sync_copy(sidx[si].at[pl.ds(base, EB)], iv_s)
                pltpu.sync_copy(didx[di].at[pl.ds(base, EB)], iv_d)
                pltpu.async_copy(srcs[slot].at[g].at[iv_s], rows, sem).wait()
                pltpu.sync_copy(rows, acc.at[iv_d], add=True)
                return carry

            lax.fori_loop(0, NBATCH, batch, 0)
            plsc.subcore_barrier()
            pltpu.sync_copy(acc.at[pl.ds(r0, stripe)],
                            outs[e].at[g].at[pl.ds(r0, stripe)])
            plsc.subcore_barrier()

        for q in range(NCHUNK // NCORE):
            gq = c * (NCHUNK // NCORE) + q
            for e, (slot, si, di, n_pad, owner) in enumerate(spec):
                if owner is None:
                    one_pass(gq, e, slot, si, di, n_pad)
                elif q == 0:
                    @pl.when(c == owner)
                    def _(e=e, slot=slot, si=si, di=di, n_pad=n_pad):
                        one_pass(0, e, slot, si, di, n_pad)

    return seg_sum


def _sage_matmul(a_parts, cnts, x_root, w, bias, *, m, relu, out_chunked,
                 root_chunked, bm=BM):
    nseg = len(a_parts)
    nk = nseg + 1
    grid = (m // bm, nk)

    def body(*refs):
        a_refs = refs[:nseg]
        c_refs = refs[nseg:2 * nseg]
        x_ref = refs[2 * nseg]
        w_ref = refs[2 * nseg + 1]
        b_ref = refs[2 * nseg + 2]
        o_ref = refs[2 * nseg + 3]
        acc = refs[2 * nseg + 4]
        k = pl.program_id(1)

        @pl.when(k == 0)
        def _():
            acc[...] = jnp.zeros_like(acc)

        for e in range(nseg):
            @pl.when(k == e)
            def _(e=e):
                scale = 1.0 / jnp.maximum(c_refs[e][0][:, 0:1], 1.0)
                for g in range(NCHUNK):
                    acc[...] += jnp.dot(
                        a_refs[e][g] * scale,
                        w_ref[g * CHUNK:(g + 1) * CHUNK, :],
                        preferred_element_type=jnp.float32)

        @pl.when(k == nseg)
        def _():
            if root_chunked:
                for g in range(NCHUNK):
                    acc[...] += jnp.dot(
                        x_ref[g], w_ref[g * CHUNK:(g + 1) * CHUNK, :],
                        preferred_element_type=jnp.float32)
            else:
                acc[...] += jnp.dot(x_ref[...], w_ref[...],
                                    preferred_element_type=jnp.float32)
            res = acc[...] + b_ref[0:1, :]
            if relu:
                res = jnp.maximum(res, 0.0)
            if out_chunked:
                for g in range(NCHUNK):
                    o_ref[g] = res[:, g * CHUNK:(g + 1) * CHUNK]
            else:
                o_ref[...] = res

    in_specs = (
        [pl.BlockSpec((NCHUNK, bm, CHUNK), lambda i, k: (0, i, 0))
         for _ in range(nseg)]
        + [pl.BlockSpec((1, bm, CHUNK), lambda i, k: (0, i, 0))
           for _ in range(nseg)]
        + [pl.BlockSpec((NCHUNK, bm, CHUNK), lambda i, k: (0, i, 0))
           if root_chunked else pl.BlockSpec((bm, HID), lambda i, k: (i, 0))]
        + [pl.BlockSpec((HID, HID), lambda i, k: (k, 0)),
           pl.BlockSpec((8, HID), lambda i, k: (0, 0))]
    )
    if out_chunked:
        out_spec = pl.BlockSpec((NCHUNK, bm, CHUNK), lambda i, k: (0, i, 0))
        out_shape = jax.ShapeDtypeStruct((NCHUNK, m, CHUNK), jnp.float32)
    else:
        out_spec = pl.BlockSpec((bm, HID), lambda i, k: (i, 0))
        out_shape = jax.ShapeDtypeStruct((m, HID), jnp.float32)

    return pl.pallas_call(
        body, grid=grid, in_specs=in_specs, out_specs=out_spec,
        out_shape=out_shape,
        scratch_shapes=[pltpu.VMEM((bm, HID), jnp.float32)],
        compiler_params=pltpu.CompilerParams(
            dimension_semantics=("parallel", "arbitrary")),
    )(*a_parts, *cnts, x_root, w, bias)


def _final_linear(x, w, bias, *, m, n_out, bm=BM):
    def body(x_ref, w_ref, b_ref, o_ref):
        o_ref[...] = jnp.dot(x_ref[...], w_ref[...],
                             preferred_element_type=jnp.float32) + b_ref[0:1, :]

    return pl.pallas_call(
        body, grid=(m // bm,),
        in_specs=[pl.BlockSpec((bm, HID), lambda i: (i, 0)),
                  pl.BlockSpec((HID, n_out), lambda i: (0, 0)),
                  pl.BlockSpec((8, n_out), lambda i: (0, 0))],
        out_specs=pl.BlockSpec((bm, n_out), lambda i: (i, 0)),
        out_shape=jax.ShapeDtypeStruct((m, n_out), jnp.float32),
        compiler_params=pltpu.CompilerParams(
            dimension_semantics=("parallel",)),
    )(x, w, bias)


def _chunked(x):
    n = x.shape[0]
    return x.reshape(n, NCHUNK, CHUNK).transpose(1, 0, 2)


def kernel(x_paper, x_author, x_institution, x_field_of_study, ei_cites,
           ei_writes, ei_rev_writes, ei_affiliated, ei_rev_affiliated,
           ei_has_topic, ei_rev_has_topic, Wl, bl, Wr, lin_W, lin_b):
    f32 = jnp.float32
    n_paper = x_paper.shape[0]
    n_author = x_author.shape[0]
    n_fos = x_field_of_study.shape[0]
    e = ei_cites.shape[1]
    pad = E_PAD - e

    def prep(ei, n_dst):
        src = jnp.concatenate([ei[0], jnp.zeros((pad,), jnp.int32)])
        dst = jnp.concatenate([ei[1], jnp.full((pad,), n_dst, jnp.int32)])
        return src, dst

    s_ci, d_ci = prep(ei_cites, n_paper)
    s_wr, d_wr = prep(ei_writes, n_paper)
    s_rw, d_rw = prep(ei_rev_writes, n_author)
    s_ra, d_ra = prep(ei_rev_affiliated, n_author)
    s_ht, d_ht = prep(ei_has_topic, n_fos)
    s_rh, d_rh = prep(ei_rev_has_topic, n_paper)
    z_idx = jnp.tile(jnp.arange(EB, dtype=jnp.int32), E_PAD // EB)

    zeros_big = jnp.zeros((BIG // NSUB, CHUNK), f32)
    ones_src = jnp.ones((1, EB, CHUNK), f32)

    xc_p = _chunked(x_paper)
    xc_a = _chunked(x_author)
    xc_i = _chunked(x_institution)
    xc_f = _chunked(x_field_of_study)
    spec1 = (
        (0, 0, 0, BIG, None),
        (1, 1, 1, BIG, None),
        (0, 2, 2, BIG, None),
        (2, 3, 3, BIG, None),
        (0, 4, 4, SMALL, None),
        (3, 5, 5, BIG, None),
        (4, 6, 0, BIG, 0),
        (4, 6, 1, BIG, 1),
        (4, 6, 2, BIG, 0),
        (4, 6, 3, BIG, 1),
        (4, 6, 4, SMALL, 0),
        (4, 6, 5, BIG, 1),
    )
    (agg_ci, agg_wr, agg_rw, agg_ra, agg_ht, agg_rh,
     cnt_ci, cnt_wr, cnt_rw, cnt_ra, cnt_ht, cnt_rh) = _make_seg_sum(
        spec1, 5, 7, 6)(xc_p, xc_a, xc_i, xc_f, ones_src,
                        s_ci, s_wr, s_rw, s_ra, s_ht, s_rh, z_idx,
                        d_ci, d_wr, d_rw, d_ra, d_ht, d_rh, zeros_big)

    def w_cat(layer, js):
        parts = [Wl[layer, j].T for j in js]
        parts.append(sum(Wr[layer, j] for j in js).T)
        return jnp.concatenate(parts, axis=0)

    def b_sum(layer, js):
        b = sum(bl[layer, j] for j in js)
        return jnp.broadcast_to(b[None, :], (8, HID))

    x1_p = _sage_matmul([agg_ci, agg_wr, agg_rh], [cnt_ci, cnt_wr, cnt_rh],
                        x_paper, w_cat(0, (0, 1, 6)), b_sum(0, (0, 1, 6)),
                        m=n_paper, relu=True, out_chunked=True,
                        root_chunked=False)
    x1_a = _sage_matmul([agg_rw, agg_ra], [cnt_rw, cnt_ra],
                        x_author, w_cat(0, (2, 4)), b_sum(0, (2, 4)),
                        m=n_author, relu=True, out_chunked=True,
                        root_chunked=False)
    x1_f = _sage_matmul([agg_ht], [cnt_ht],
                        x_field_of_study, w_cat(0, (5,)), b_sum(0, (5,)),
                        m=n_fos, relu=True, out_chunked=True,
                        root_chunked=False)

    spec2 = ((0, 0, 0, BIG, None), (1, 1, 1, BIG, None),
             (2, 2, 2, BIG, None))
    agg2_ci, agg2_wr, agg2_rh = _make_seg_sum(spec2, 3, 3, 3)(
        x1_p, x1_a, x1_f, s_ci, s_wr, s_rh, d_ci, d_wr, d_rh, zeros_big)

    x2_p = _sage_matmul([agg2_ci, agg2_wr, agg2_rh],
                        [cnt_ci, cnt_wr, cnt_rh],
                        x1_p, w_cat(1, (0, 1, 6)), b_sum(1, (0, 1, 6)),
                        m=n_paper, relu=True, out_chunked=False,
                        root_chunked=True)

    n_out = lin_W.shape[0]
    lin_bias = jnp.broadcast_to(lin_b[None, :], (8, n_out))
    return _final_linear(x2_p, lin_W.T, lin_bias, m=n_paper, n_out=n_out)


# speedup vs baseline: 4.3333x; 1.1828x over previous
import functools

import jax
import jax.numpy as jnp
from jax import lax
from jax.experimental import pallas as pl
from jax.experimental.pallas import tpu as pltpu
from jax.experimental.pallas import tpu_sc as plsc

HID = 512
CHUNK = 128
NCHUNK = HID // CHUNK
NSUB = 16
NCORE = 2
EB = 128
NBATCH = 10
E_PAD = NSUB * NBATCH * EB
BIG = 10240
SMALL = 1024
BM = 1000


def _make_seg_sum(spec, n_slots, n_sidx, n_didx):
    n_et = len(spec)
    mesh = plsc.VectorSubcoreMesh(core_axis_name="c", subcore_axis_name="s")
    out_type = [jax.ShapeDtypeStruct((NCHUNK, np_, CHUNK), jnp.float32)
                for _, _, _, np_, _ in spec]
    scratch = [
        pltpu.VMEM((NBATCH, EB), jnp.int32),
        pltpu.VMEM((NBATCH, EB), jnp.int32),
        pltpu.VMEM((EB, CHUNK), jnp.float32),
        pltpu.VMEM((EB, CHUNK), jnp.float32),
        pltpu.VMEM_SHARED((BIG, CHUNK), jnp.float32),
        pltpu.SemaphoreType.DMA,
    ]

    @functools.partial(pl.kernel, mesh=mesh, out_type=out_type,
                       scratch_types=scratch)
    def seg_sum(*refs):
        srcs = refs[:n_slots]
        sidx = refs[n_slots:n_slots + n_sidx]
        didx = refs[n_slots + n_sidx:n_slots + n_sidx + n_didx]
        zeros_h = refs[n_slots + n_sidx + n_didx]
        outs = refs[n_slots + n_sidx + n_didx + 1:
                    n_slots + n_sidx + n_didx + 1 + n_et]
        (iv_s, iv_d, rows0, rows1, acc,
         sem) = refs[n_slots + n_sidx + n_didx + 1 + n_et:]
        c = lax.axis_index("c")
        s = lax.axis_index("s")

        def one_pass(g, e, slot, si, di, n_pad):
            stripe = n_pad // NSUB
            r0 = s * stripe
            pltpu.sync_copy(sidx[si].at[s], iv_s)
            pltpu.sync_copy(didx[di].at[s], iv_d)
            pltpu.sync_copy(zeros_h.at[pl.ds(0, stripe)],
                            acc.at[pl.ds(r0, stripe)])
            plsc.subcore_barrier()

            bufs = (rows0, rows1)

            def gather(j, buf):
                return pltpu.async_copy(srcs[slot].at[g].at[iv_s.at[j]],
                                        buf, sem)

            desc = [gather(0, rows0), None]
            for j in range(NBATCH):
                desc[j % 2].wait()
                if j + 1 < NBATCH:
                    desc[(j + 1) % 2] = gather(j + 1, bufs[(j + 1) % 2])
                pltpu.sync_copy(bufs[j % 2], acc.at[iv_d.at[j]], add=True)

            plsc.subcore_barrier()
            pltpu.sync_copy(acc.at[pl.ds(r0, stripe)],
                            outs[e].at[g].at[pl.ds(r0, stripe)])
            plsc.subcore_barrier()

        for q in range(NCHUNK // NCORE):
            gq = c * (NCHUNK // NCORE) + q
            for e, (slot, si, di, n_pad, owner) in enumerate(spec):
                if owner is None:
                    one_pass(gq, e, slot, si, di, n_pad)
                elif q == 0:
                    @pl.when(c == owner)
                    def _(e=e, slot=slot, si=si, di=di, n_pad=n_pad):
                        one_pass(0, e, slot, si, di, n_pad)

    return seg_sum


def _sage_matmul(a_parts, cnts, x_root, w, bias, *, m, relu, out_chunked,
                 root_chunked, bm=BM):
    nseg = len(a_parts)
    nk = nseg + 1
    grid = (m // bm, nk)

    def body(*refs):
        a_refs = refs[:nseg]
        c_refs = refs[nseg:2 * nseg]
        x_ref = refs[2 * nseg]
        w_ref = refs[2 * nseg + 1]
        b_ref = refs[2 * nseg + 2]
        o_ref = refs[2 * nseg + 3]
        acc = refs[2 * nseg + 4]
        k = pl.program_id(1)

        @pl.when(k == 0)
        def _():
            acc[...] = jnp.zeros_like(acc)

        for e in range(nseg):
            @pl.when(k == e)
            def _(e=e):
                scale = 1.0 / jnp.maximum(c_refs[e][0][:, 0:1], 1.0)
                for g in range(NCHUNK):
                    acc[...] += jnp.dot(
                        a_refs[e][g] * scale,
                        w_ref[g * CHUNK:(g + 1) * CHUNK, :],
                        preferred_element_type=jnp.float32)

        @pl.when(k == nseg)
        def _():
            if root_chunked:
                for g in range(NCHUNK):
                    acc[...] += jnp.dot(
                        x_ref[g], w_ref[g * CHUNK:(g + 1) * CHUNK, :],
                        preferred_element_type=jnp.float32)
            else:
                acc[...] += jnp.dot(x_ref[...], w_ref[...],
                                    preferred_element_type=jnp.float32)
            res = acc[...] + b_ref[0:1, :]
            if relu:
                res = jnp.maximum(res, 0.0)
            if out_chunked:
                for g in range(NCHUNK):
                    o_ref[g] = res[:, g * CHUNK:(g + 1) * CHUNK]
            else:
                o_ref[...] = res

    in_specs = (
        [pl.BlockSpec((NCHUNK, bm, CHUNK), lambda i, k: (0, i, 0))
         for _ in range(nseg)]
        + [pl.BlockSpec((1, bm, CHUNK), lambda i, k: (0, i, 0))
           for _ in range(nseg)]
        + [pl.BlockSpec((NCHUNK, bm, CHUNK), lambda i, k: (0, i, 0))
           if root_chunked else pl.BlockSpec((bm, HID), lambda i, k: (i, 0))]
        + [pl.BlockSpec((HID, HID), lambda i, k: (k, 0)),
           pl.BlockSpec((8, HID), lambda i, k: (0, 0))]
    )
    if out_chunked:
        out_spec = pl.BlockSpec((NCHUNK, bm, CHUNK), lambda i, k: (0, i, 0))
        out_shape = jax.ShapeDtypeStruct((NCHUNK, m, CHUNK), jnp.float32)
    else:
        out_spec = pl.BlockSpec((bm, HID), lambda i, k: (i, 0))
        out_shape = jax.ShapeDtypeStruct((m, HID), jnp.float32)

    return pl.pallas_call(
        body, grid=grid, in_specs=in_specs, out_specs=out_spec,
        out_shape=out_shape,
        scratch_shapes=[pltpu.VMEM((bm, HID), jnp.float32)],
        compiler_params=pltpu.CompilerParams(
            dimension_semantics=("parallel", "arbitrary")),
    )(*a_parts, *cnts, x_root, w, bias)


def _final_linear(x, w, bias, *, m, n_out, bm=BM):
    def body(x_ref, w_ref, b_ref, o_ref):
        o_ref[...] = jnp.dot(x_ref[...], w_ref[...],
                             preferred_element_type=jnp.float32) + b_ref[0:1, :]

    return pl.pallas_call(
        body, grid=(m // bm,),
        in_specs=[pl.BlockSpec((bm, HID), lambda i: (i, 0)),
                  pl.BlockSpec((HID, n_out), lambda i: (0, 0)),
                  pl.BlockSpec((8, n_out), lambda i: (0, 0))],
        out_specs=pl.BlockSpec((bm, n_out), lambda i: (i, 0)),
        out_shape=jax.ShapeDtypeStruct((m, n_out), jnp.float32),
        compiler_params=pltpu.CompilerParams(
            dimension_semantics=("parallel",)),
    )(x, w, bias)


def _chunked(x):
    n = x.shape[0]
    return x.reshape(n, NCHUNK, CHUNK).transpose(1, 0, 2)


def kernel(x_paper, x_author, x_institution, x_field_of_study, ei_cites,
           ei_writes, ei_rev_writes, ei_affiliated, ei_rev_affiliated,
           ei_has_topic, ei_rev_has_topic, Wl, bl, Wr, lin_W, lin_b):
    f32 = jnp.float32
    n_paper = x_paper.shape[0]
    n_author = x_author.shape[0]
    n_fos = x_field_of_study.shape[0]
    e = ei_cites.shape[1]
    pad = E_PAD - e

    def prep(ei, n_dst):
        src = jnp.concatenate([ei[0], jnp.zeros((pad,), jnp.int32)])
        dst = jnp.concatenate([ei[1], jnp.full((pad,), n_dst, jnp.int32)])
        return (src.reshape(NSUB, NBATCH, EB),
                dst.reshape(NSUB, NBATCH, EB))

    s_ci, d_ci = prep(ei_cites, n_paper)
    s_wr, d_wr = prep(ei_writes, n_paper)
    s_rw, d_rw = prep(ei_rev_writes, n_author)
    s_ra, d_ra = prep(ei_rev_affiliated, n_author)
    s_ht, d_ht = prep(ei_has_topic, n_fos)
    s_rh, d_rh = prep(ei_rev_has_topic, n_paper)
    z_idx = jnp.broadcast_to(jnp.arange(EB, dtype=jnp.int32),
                             (NSUB, NBATCH, EB))

    zeros_big = jnp.zeros((BIG // NSUB, CHUNK), f32)
    ones_src = jnp.ones((1, EB, CHUNK), f32)

    xc_p = _chunked(x_paper)
    xc_a = _chunked(x_author)
    xc_i = _chunked(x_institution)
    xc_f = _chunked(x_field_of_study)
    spec1 = (
        (0, 0, 0, BIG, None),
        (1, 1, 1, BIG, None),
        (0, 2, 2, BIG, None),
        (2, 3, 3, BIG, None),
        (0, 4, 4, SMALL, None),
        (3, 5, 5, BIG, None),
        (4, 6, 0, BIG, 0),
        (4, 6, 1, BIG, 1),
        (4, 6, 2, BIG, 0),
        (4, 6, 3, BIG, 1),
        (4, 6, 4, SMALL, 0),
        (4, 6, 5, BIG, 1),
    )
    (agg_ci, agg_wr, agg_rw, agg_ra, agg_ht, agg_rh,
     cnt_ci, cnt_wr, cnt_rw, cnt_ra, cnt_ht, cnt_rh) = _make_seg_sum(
        spec1, 5, 7, 6)(xc_p, xc_a, xc_i, xc_f, ones_src,
                        s_ci, s_wr, s_rw, s_ra, s_ht, s_rh, z_idx,
                        d_ci, d_wr, d_rw, d_ra, d_ht, d_rh, zeros_big)

    def w_cat(layer, js):
        parts = [Wl[layer, j].T for j in js]
        parts.append(sum(Wr[layer, j] for j in js).T)
        return jnp.concatenate(parts, axis=0)

    def b_sum(layer, js):
        b = sum(bl[layer, j] for j in js)
        return jnp.broadcast_to(b[None, :], (8, HID))

    x1_p = _sage_matmul([agg_ci, agg_wr, agg_rh], [cnt_ci, cnt_wr, cnt_rh],
                        x_paper, w_cat(0, (0, 1, 6)), b_sum(0, (0, 1, 6)),
                        m=n_paper, relu=True, out_chunked=True,
                        root_chunked=False)
    x1_a = _sage_matmul([agg_rw, agg_ra], [cnt_rw, cnt_ra],
                        x_author, w_cat(0, (2, 4)), b_sum(0, (2, 4)),
                        m=n_author, relu=True, out_chunked=True,
                        root_chunked=False)
    x1_f = _sage_matmul([agg_ht], [cnt_ht],
                        x_field_of_study, w_cat(0, (5,)), b_sum(0, (5,)),
                        m=n_fos, relu=True, out_chunked=True,
                        root_chunked=False)

    spec2 = ((0, 0, 0, BIG, None), (1, 1, 1, BIG, None),
             (2, 2, 2, BIG, None))
    agg2_ci, agg2_wr, agg2_rh = _make_seg_sum(spec2, 3, 3, 3)(
        x1_p, x1_a, x1_f, s_ci, s_wr, s_rh, d_ci, d_wr, d_rh, zeros_big)

    x2_p = _sage_matmul([agg2_ci, agg2_wr, agg2_rh],
                        [cnt_ci, cnt_wr, cnt_rh],
                        x1_p, w_cat(1, (0, 1, 6)), b_sum(1, (0, 1, 6)),
                        m=n_paper, relu=True, out_chunked=False,
                        root_chunked=True)

    n_out = lin_W.shape[0]
    lin_bias = jnp.broadcast_to(lin_b[None, :], (8, n_out))
    return _final_linear(x2_p, lin_W.T, lin_bias, m=n_paper, n_out=n_out)


# speedup vs baseline: 4.3493x; 1.0037x over previous
import functools

import jax
import jax.numpy as jnp
from jax import lax
from jax.experimental import pallas as pl
from jax.experimental.pallas import tpu as pltpu
from jax.experimental.pallas import tpu_sc as plsc

HID = 512
CHUNK = 128
NCHUNK = HID // CHUNK
NSUB = 16
NCORE = 2
EB = 128
NBATCH = 10
E_PAD = NSUB * NBATCH * EB
BIG = 10240
SMALL = 1024
BM = 1000


def _make_seg_sum(spec, n_slots, n_sidx, n_didx):
    n_et = len(spec)
    mesh = plsc.VectorSubcoreMesh(core_axis_name="c", subcore_axis_name="s")
    out_type = [jax.ShapeDtypeStruct((NCHUNK, np_, CHUNK), jnp.float32)
                for _, _, _, np_, _ in spec]
    scratch = [
        pltpu.VMEM((NBATCH, EB), jnp.int32),
        pltpu.VMEM((NBATCH, EB), jnp.int32),
        pltpu.VMEM((2, EB, CHUNK), jnp.float32),
        pltpu.VMEM_SHARED((BIG, CHUNK), jnp.float32),
        pltpu.SemaphoreType.DMA,
    ]

    @functools.partial(pl.kernel, mesh=mesh, out_type=out_type,
                       scratch_types=scratch)
    def seg_sum(*refs):
        srcs = refs[:n_slots]
        sidx = refs[n_slots:n_slots + n_sidx]
        didx = refs[n_slots + n_sidx:n_slots + n_sidx + n_didx]
        zeros_h = refs[n_slots + n_sidx + n_didx]
        outs = refs[n_slots + n_sidx + n_didx + 1:
                    n_slots + n_sidx + n_didx + 1 + n_et]
        (iv_s, iv_d, ring, acc,
         sem_g) = refs[n_slots + n_sidx + n_didx + 1 + n_et:]
        c = lax.axis_index("c")
        s = lax.axis_index("s")

        def one_pass(g, e, slot, si, di, n_pad):
            stripe = n_pad // NSUB
            r0 = s * stripe
            pltpu.sync_copy(sidx[si].at[s], iv_s)
            pltpu.sync_copy(didx[di].at[s], iv_d)
            pltpu.sync_copy(zeros_h.at[pl.ds(0, stripe)],
                            acc.at[pl.ds(r0, stripe)])
            plsc.subcore_barrier()

            def gather(j):
                return pltpu.async_copy(srcs[slot].at[g].at[iv_s.at[j]],
                                        ring.at[j % 2], sem_g)

            desc = [gather(0), None]
            for j in range(NBATCH):
                desc[j % 2].wait()
                if j + 1 < NBATCH:
                    desc[(j + 1) % 2] = gather(j + 1)
                pltpu.sync_copy(ring.at[j % 2], acc.at[iv_d.at[j]], add=True)

            plsc.subcore_barrier()
            pltpu.sync_copy(acc.at[pl.ds(r0, stripe)],
                            outs[e].at[g].at[pl.ds(r0, stripe)])
            plsc.subcore_barrier()

        for q in range(NCHUNK // NCORE):
            gq = c * (NCHUNK // NCORE) + q
            for e, (slot, si, di, n_pad, owner) in enumerate(spec):
                if owner is None:
                    one_pass(gq, e, slot, si, di, n_pad)
                elif q == 0:
                    @pl.when(c == owner)
                    def _(e=e, slot=slot, si=si, di=di, n_pad=n_pad):
                        one_pass(0, e, slot, si, di, n_pad)

    return seg_sum


def _sage_matmul(a_parts, cnts, x_root, w, bias, *, m, relu, out_chunked,
                 root_chunked, bm=BM):
    nseg = len(a_parts)
    nk = nseg + 1
    grid = (m // bm, nk)

    def body(*refs):
        a_refs = refs[:nseg]
        c_refs = refs[nseg:2 * nseg]
        x_ref = refs[2 * nseg]
        w_ref = refs[2 * nseg + 1]
        b_ref = refs[2 * nseg + 2]
        o_ref = refs[2 * nseg + 3]
        acc = refs[2 * nseg + 4]
        k = pl.program_id(1)

        @pl.when(k == 0)
        def _():
            acc[...] = jnp.zeros_like(acc)

        for e in range(nseg):
            @pl.when(k == e)
            def _(e=e):
                scale = 1.0 / jnp.maximum(c_refs[e][0][:, 0:1], 1.0)
                for g in range(NCHUNK):
                    acc[...] += jnp.dot(
                        a_refs[e][g] * scale,
                        w_ref[g * CHUNK:(g + 1) * CHUNK, :],
                        preferred_element_type=jnp.float32)

        @pl.when(k == nseg)
        def _():
            if root_chunked:
                for g in range(NCHUNK):
                    acc[...] += jnp.dot(
                        x_ref[g], w_ref[g * CHUNK:(g + 1) * CHUNK, :],
                        preferred_element_type=jnp.float32)
            else:
                acc[...] += jnp.dot(x_ref[...], w_ref[...],
                                    preferred_element_type=jnp.float32)
            res = acc[...] + b_ref[0:1, :]
            if relu:
                res = jnp.maximum(res, 0.0)
            if out_chunked:
                for g in range(NCHUNK):
                    o_ref[g] = res[:, g * CHUNK:(g + 1) * CHUNK]
            else:
                o_ref[...] = res

    in_specs = (
        [pl.BlockSpec((NCHUNK, bm, CHUNK), lambda i, k: (0, i, 0))
         for _ in range(nseg)]
        + [pl.BlockSpec((1, bm, CHUNK), lambda i, k: (0, i, 0))
           for _ in range(nseg)]
        + [pl.BlockSpec((NCHUNK, bm, CHUNK), lambda i, k: (0, i, 0))
           if root_chunked else pl.BlockSpec((bm, HID), lambda i, k: (i, 0))]
        + [pl.BlockSpec((HID, HID), lambda i, k: (k, 0)),
           pl.BlockSpec((8, HID), lambda i, k: (0, 0))]
    )
    if out_chunked:
        out_spec = pl.BlockSpec((NCHUNK, bm, CHUNK), lambda i, k: (0, i, 0))
        out_shape = jax.ShapeDtypeStruct((NCHUNK, m, CHUNK), jnp.float32)
    else:
        out_spec = pl.BlockSpec((bm, HID), lambda i, k: (i, 0))
        out_shape = jax.ShapeDtypeStruct((m, HID), jnp.float32)

    return pl.pallas_call(
        body, grid=grid, in_specs=in_specs, out_specs=out_spec,
        out_shape=out_shape,
        scratch_shapes=[pltpu.VMEM((bm, HID), jnp.float32)],
        compiler_params=pltpu.CompilerParams(
            dimension_semantics=("parallel", "arbitrary")),
    )(*a_parts, *cnts, x_root, w, bias)


def _final_linear(x, w, bias, *, m, n_out, bm=BM):
    def body(x_ref, w_ref, b_ref, o_ref):
        o_ref[...] = jnp.dot(x_ref[...], w_ref[...],
                             preferred_element_type=jnp.float32) + b_ref[0:1, :]

    return pl.pallas_call(
        body, grid=(m // bm,),
        in_specs=[pl.BlockSpec((bm, HID), lambda i: (i, 0)),
                  pl.BlockSpec((HID, n_out), lambda i: (0, 0)),
                  pl.BlockSpec((8, n_out), lambda i: (0, 0))],
        out_specs=pl.BlockSpec((bm, n_out), lambda i: (i, 0)),
        out_shape=jax.ShapeDtypeStruct((m, n_out), jnp.float32),
        compiler_params=pltpu.CompilerParams(
            dimension_semantics=("parallel",)),
    )(x, w, bias)


def _chunked(x):
    n = x.shape[0]
    return x.reshape(n, NCHUNK, CHUNK).transpose(1, 0, 2)


def kernel(x_paper, x_author, x_institution, x_field_of_study, ei_cites,
           ei_writes, ei_rev_writes, ei_affiliated, ei_rev_affiliated,
           ei_has_topic, ei_rev_has_topic, Wl, bl, Wr, lin_W, lin_b):
    f32 = jnp.float32
    n_paper = x_paper.shape[0]
    n_author = x_author.shape[0]
    n_fos = x_field_of_study.shape[0]
    e = ei_cites.shape[1]
    pad = E_PAD - e

    def prep(ei, n_dst):
        src = jnp.concatenate([ei[0], jnp.zeros((pad,), jnp.int32)])
        dst = jnp.concatenate([ei[1], jnp.full((pad,), n_dst, jnp.int32)])
        return (src.reshape(NSUB, NBATCH, EB),
                dst.reshape(NSUB, NBATCH, EB))

    s_ci, d_ci = prep(ei_cites, n_paper)
    s_wr, d_wr = prep(ei_writes, n_paper)
    s_rw, d_rw = prep(ei_rev_writes, n_author)
    s_ra, d_ra = prep(ei_rev_affiliated, n_author)
    s_ht, d_ht = prep(ei_has_topic, n_fos)
    s_rh, d_rh = prep(ei_rev_has_topic, n_paper)
    z_idx = jnp.broadcast_to(jnp.arange(EB, dtype=jnp.int32),
                             (NSUB, NBATCH, EB))

    zeros_big = jnp.zeros((BIG // NSUB, CHUNK), f32)
    ones_src = jnp.ones((1, EB, CHUNK), f32)

    xc_p = _chunked(x_paper)
    xc_a = _chunked(x_author)
    xc_i = _chunked(x_institution)
    xc_f = _chunked(x_field_of_study)
    spec_a = (
        (0, 0, 0, BIG, None),
        (1, 1, 1, BIG, None),
        (2, 2, 2, BIG, None),
        (3, 3, 0, BIG, 0),
        (3, 3, 1, BIG, 1),
        (3, 3, 2, BIG, 0),
    )
    (agg_ci, agg_wr, agg_rh, cnt_ci, cnt_wr, cnt_rh) = _make_seg_sum(
        spec_a, 4, 4, 3)(xc_p, xc_a, xc_f, ones_src,
                         s_ci, s_wr, s_rh, z_idx,
                         d_ci, d_wr, d_rh, zeros_big)
    spec_b = (
        (0, 0, 0, BIG, None),
        (1, 1, 1, BIG, None),
        (0, 2, 2, SMALL, None),
        (2, 3, 0, BIG, 1),
        (2, 3, 1, BIG, 0),
        (2, 3, 2, SMALL, 1),
    )
    (agg_rw, agg_ra, agg_ht, cnt_rw, cnt_ra, cnt_ht) = _make_seg_sum(
        spec_b, 3, 4, 3)(xc_p, xc_i, ones_src,
                         s_rw, s_ra, s_ht, z_idx,
                         d_rw, d_ra, d_ht, zeros_big)

    def w_cat(layer, js):
        parts = [Wl[layer, j].T for j in js]
        parts.append(sum(Wr[layer, j] for j in js).T)
        return jnp.concatenate(parts, axis=0)

    def b_sum(layer, js):
        b = sum(bl[layer, j] for j in js)
        return jnp.broadcast_to(b[None, :], (8, HID))

    x1_p = _sage_matmul([agg_ci, agg_wr, agg_rh], [cnt_ci, cnt_wr, cnt_rh],
                        x_paper, w_cat(0, (0, 1, 6)), b_sum(0, (0, 1, 6)),
                        m=n_paper, relu=True, out_chunked=True,
                        root_chunked=False)
    x1_a = _sage_matmul([agg_rw, agg_ra], [cnt_rw, cnt_ra],
                        x_author, w_cat(0, (2, 4)), b_sum(0, (2, 4)),
                        m=n_author, relu=True, out_chunked=True,
                        root_chunked=False)
    x1_f = _sage_matmul([agg_ht], [cnt_ht],
                        x_field_of_study, w_cat(0, (5,)), b_sum(0, (5,)),
                        m=n_fos, relu=True, out_chunked=True,
                        root_chunked=False)

    spec2 = ((0, 0, 0, BIG, None), (1, 1, 1, BIG, None),
             (2, 2, 2, BIG, None))
    agg2_ci, agg2_wr, agg2_rh = _make_seg_sum(spec2, 3, 3, 3)(
        x1_p, x1_a, x1_f, s_ci, s_wr, s_rh, d_ci, d_wr, d_rh, zeros_big)

    x2_p = _sage_matmul([agg2_ci, agg2_wr, agg2_rh],
                        [cnt_ci, cnt_wr, cnt_rh],
                        x1_p, w_cat(1, (0, 1, 6)), b_sum(1, (0, 1, 6)),
                        m=n_paper, relu=True, out_chunked=False,
                        root_chunked=True)

    n_out = lin_W.shape[0]
    lin_bias = jnp.broadcast_to(lin_b[None, :], (8, n_out))
    return _final_linear(x2_p, lin_W.T, lin_bias, m=n_paper, n_out=n_out)


# speedup vs baseline: 4.3529x; 1.0008x over previous
import functools

import jax
import jax.numpy as jnp
from jax import lax
from jax.experimental import pallas as pl
from jax.experimental.pallas import tpu as pltpu
from jax.experimental.pallas import tpu_sc as plsc

HID = 512
CHUNK = 128
NCHUNK = HID // CHUNK
NSUB = 16
NCORE = 2
EB = 128
NBATCH = 10
E_PAD = NSUB * NBATCH * EB
BIG = 10240
SMALL = 1024
BM = 1000


def _make_seg_sum(spec, n_slots, n_sidx, n_didx):
    n_et = len(spec)
    mesh = plsc.VectorSubcoreMesh(core_axis_name="c", subcore_axis_name="s")
    out_type = [jax.ShapeDtypeStruct((NCHUNK, np_, CHUNK), jnp.float32)
                for _, _, _, np_, _ in spec]
    scratch = [
        pltpu.VMEM((NBATCH, EB), jnp.int32),
        pltpu.VMEM((NBATCH, EB), jnp.int32),
        pltpu.VMEM((2, EB, CHUNK), jnp.float32),
        pltpu.VMEM_SHARED((BIG, CHUNK), jnp.float32),
        pltpu.SemaphoreType.DMA,
    ]

    @functools.partial(pl.kernel, mesh=mesh, out_type=out_type,
                       scratch_types=scratch)
    def seg_sum(*refs):
        srcs = refs[:n_slots]
        sidx = refs[n_slots:n_slots + n_sidx]
        didx = refs[n_slots + n_sidx:n_slots + n_sidx + n_didx]
        zeros_h = refs[n_slots + n_sidx + n_didx]
        outs = refs[n_slots + n_sidx + n_didx + 1:
                    n_slots + n_sidx + n_didx + 1 + n_et]
        (iv_s, iv_d, ring, acc,
         sem_g) = refs[n_slots + n_sidx + n_didx + 1 + n_et:]
        c = lax.axis_index("c")
        s = lax.axis_index("s")

        def one_pass(table, rng, ac, zh, out_v, si, di, n_pad):
            stripe = n_pad // NSUB
            r0 = s * stripe
            pltpu.sync_copy(sidx[si].at[s], iv_s)
            pltpu.sync_copy(didx[di].at[s], iv_d)
            pltpu.sync_copy(zh.at[pl.ds(0, stripe)],
                            ac.at[pl.ds(r0, stripe)])
            plsc.subcore_barrier()

            def gather(j):
                return pltpu.async_copy(table.at[iv_s.at[j]],
                                        rng.at[j % 2], sem_g)

            desc = [gather(0), None]
            for j in range(NBATCH):
                desc[j % 2].wait()
                if j + 1 < NBATCH:
                    desc[(j + 1) % 2] = gather(j + 1)
                pltpu.sync_copy(rng.at[j % 2], ac.at[iv_d.at[j]], add=True)

            plsc.subcore_barrier()
            pltpu.sync_copy(ac.at[pl.ds(r0, stripe)],
                            out_v.at[pl.ds(r0, stripe)])
            plsc.subcore_barrier()

        for q in range(NCHUNK // NCORE):
            gq = c * (NCHUNK // NCORE) + q
            for e, (slot, si, di, n_pad, owner) in enumerate(spec):
                if owner is None:
                    one_pass(srcs[slot].at[gq], ring, acc, zeros_h,
                             outs[e].at[gq], si, di, n_pad)
                elif q == 0:
                    @pl.when(c == owner)
                    def _(e=e, slot=slot, si=si, di=di, n_pad=n_pad):
                        one_pass(srcs[slot].at[0], ring, acc, zeros_h,
                                 outs[e].at[0], si, di, n_pad)

    return seg_sum


def _sage_matmul(a_parts, cnts, x_root, w, bias, *, m, relu, out_chunked,
                 root_chunked, bm=BM):
    nseg = len(a_parts)
    nk = nseg + 1
    grid = (m // bm, nk)

    def body(*refs):
        a_refs = refs[:nseg]
        c_refs = refs[nseg:2 * nseg]
        x_ref = refs[2 * nseg]
        w_ref = refs[2 * nseg + 1]
        b_ref = refs[2 * nseg + 2]
        o_ref = refs[2 * nseg + 3]
        acc = refs[2 * nseg + 4]
        k = pl.program_id(1)

        @pl.when(k == 0)
        def _():
            acc[...] = jnp.zeros_like(acc)

        for e in range(nseg):
            @pl.when(k == e)
            def _(e=e):
                scale = 1.0 / jnp.maximum(c_refs[e][0][:, 0:1], 1.0)
                for g in range(NCHUNK):
                    acc[...] += jnp.dot(
                        a_refs[e][g] * scale,
                        w_ref[g * CHUNK:(g + 1) * CHUNK, :],
                        preferred_element_type=jnp.float32)

        @pl.when(k == nseg)
        def _():
            if root_chunked:
                for g in range(NCHUNK):
                    acc[...] += jnp.dot(
                        x_ref[g], w_ref[g * CHUNK:(g + 1) * CHUNK, :],
                        preferred_element_type=jnp.float32)
            else:
                acc[...] += jnp.dot(x_ref[...], w_ref[...],
                                    preferred_element_type=jnp.float32)
            res = acc[...] + b_ref[0:1, :]
            if relu:
                res = jnp.maximum(res, 0.0)
            if out_chunked:
                for g in range(NCHUNK):
                    o_ref[g] = res[:, g * CHUNK:(g + 1) * CHUNK]
            else:
                o_ref[...] = res

    in_specs = (
        [pl.BlockSpec((NCHUNK, bm, CHUNK), lambda i, k: (0, i, 0))
         for _ in range(nseg)]
        + [pl.BlockSpec((1, bm, CHUNK), lambda i, k: (0, i, 0))
           for _ in range(nseg)]
        + [pl.BlockSpec((NCHUNK, bm, CHUNK), lambda i, k: (0, i, 0))
           if root_chunked else pl.BlockSpec((bm, HID), lambda i, k: (i, 0))]
        + [pl.BlockSpec((HID, HID), lambda i, k: (k, 0)),
           pl.BlockSpec((8, HID), lambda i, k: (0, 0))]
    )
    if out_chunked:
        out_spec = pl.BlockSpec((NCHUNK, bm, CHUNK), lambda i, k: (0, i, 0))
        out_shape = jax.ShapeDtypeStruct((NCHUNK, m, CHUNK), jnp.float32)
    else:
        out_spec = pl.BlockSpec((bm, HID), lambda i, k: (i, 0))
        out_shape = jax.ShapeDtypeStruct((m, HID), jnp.float32)

    return pl.pallas_call(
        body, grid=grid, in_specs=in_specs, out_specs=out_spec,
        out_shape=out_shape,
        scratch_shapes=[pltpu.VMEM((bm, HID), jnp.float32)],
        compiler_params=pltpu.CompilerParams(
            dimension_semantics=("parallel", "arbitrary")),
    )(*a_parts, *cnts, x_root, w, bias)


def _final_linear(x, w, bias, *, m, n_out, bm=BM):
    def body(x_ref, w_ref, b_ref, o_ref):
        o_ref[...] = jnp.dot(x_ref[...], w_ref[...],
                             preferred_element_type=jnp.float32) + b_ref[0:1, :]

    return pl.pallas_call(
        body, grid=(m // bm,),
        in_specs=[pl.BlockSpec((bm, HID), lambda i: (i, 0)),
                  pl.BlockSpec((HID, n_out), lambda i: (0, 0)),
                  pl.BlockSpec((8, n_out), lambda i: (0, 0))],
        out_specs=pl.BlockSpec((bm, n_out), lambda i: (i, 0)),
        out_shape=jax.ShapeDtypeStruct((m, n_out), jnp.float32),
        compiler_params=pltpu.CompilerParams(
            dimension_semantics=("parallel",)),
    )(x, w, bias)


def _chunked(x):
    n = x.shape[0]
    return x.reshape(n, NCHUNK, CHUNK).transpose(1, 0, 2)


def kernel(x_paper, x_author, x_institution, x_field_of_study, ei_cites,
           ei_writes, ei_rev_writes, ei_affiliated, ei_rev_affiliated,
           ei_has_topic, ei_rev_has_topic, Wl, bl, Wr, lin_W, lin_b):
    f32 = jnp.float32
    n_paper = x_paper.shape[0]
    n_author = x_author.shape[0]
    n_fos = x_field_of_study.shape[0]
    e = ei_cites.shape[1]
    pad = E_PAD - e

    def prep(ei, n_dst):
        src = jnp.concatenate([ei[0], jnp.zeros((pad,), jnp.int32)])
        dst = jnp.concatenate([ei[1], jnp.full((pad,), n_dst, jnp.int32)])
        return (src.reshape(NSUB, NBATCH, EB),
                dst.reshape(NSUB, NBATCH, EB))

    s_ci, d_ci = prep(ei_cites, n_paper)
    s_wr, d_wr = prep(ei_writes, n_paper)
    s_rw, d_rw = prep(ei_rev_writes, n_author)
    s_ra, d_ra = prep(ei_rev_affiliated, n_author)
    s_ht, d_ht = prep(ei_has_topic, n_fos)
    s_rh, d_rh = prep(ei_rev_has_topic, n_paper)
    z_idx = jnp.broadcast_to(jnp.arange(EB, dtype=jnp.int32),
                             (NSUB, NBATCH, EB))

    zeros_big = jnp.zeros((BIG // NSUB, CHUNK), f32)
    ones_src = jnp.ones((1, EB, CHUNK), f32)

    xc_p = _chunked(x_paper)
    xc_a = _chunked(x_author)
    xc_i = _chunked(x_institution)
    xc_f = _chunked(x_field_of_study)
    spec_a = (
        (0, 0, 0, BIG, None),
        (1, 1, 1, BIG, None),
        (2, 2, 2, BIG, None),
        (3, 3, 0, BIG, 0),
        (3, 3, 1, BIG, 1),
        (3, 3, 2, BIG, 0),
    )
    (agg_ci, agg_wr, agg_rh, cnt_ci, cnt_wr, cnt_rh) = _make_seg_sum(
        spec_a, 4, 4, 3)(xc_p, xc_a, xc_f, ones_src,
                         s_ci, s_wr, s_rh, z_idx,
                         d_ci, d_wr, d_rh, zeros_big)
    spec_b = (
        (0, 0, 0, BIG, None),
        (1, 1, 1, BIG, None),
        (0, 2, 2, SMALL, None),
        (2, 3, 0, BIG, 1),
        (2, 3, 1, BIG, 0),
        (2, 3, 2, SMALL, 1),
    )
    (agg_rw, agg_ra, agg_ht, cnt_rw, cnt_ra, cnt_ht) = _make_seg_sum(
        spec_b, 3, 4, 3)(xc_p, xc_i, ones_src,
                         s_rw, s_ra, s_ht, z_idx,
                         d_rw, d_ra, d_ht, zeros_big)

    def w_cat(layer, js):
        parts = [Wl[layer, j].T for j in js]
        parts.append(sum(Wr[layer, j] for j in js).T)
        return jnp.concatenate(parts, axis=0)

    def b_sum(layer, js):
        b = sum(bl[layer, j] for j in js)
        return jnp.broadcast_to(b[None, :], (8, HID))

    x1_p = _sage_matmul([agg_ci, agg_wr, agg_rh], [cnt_ci, cnt_wr, cnt_rh],
                        x_paper, w_cat(0, (0, 1, 6)), b_sum(0, (0, 1, 6)),
                        m=n_paper, relu=True, out_chunked=True,
                        root_chunked=False)
    x1_a = _sage_matmul([agg_rw, agg_ra], [cnt_rw, cnt_ra],
                        x_author, w_cat(0, (2, 4)), b_sum(0, (2, 4)),
                        m=n_author, relu=True, out_chunked=True,
                        root_chunked=False)
    x1_f = _sage_matmul([agg_ht], [cnt_ht],
                        x_field_of_study, w_cat(0, (5,)), b_sum(0, (5,)),
                        m=n_fos, relu=True, out_chunked=True,
                        root_chunked=False)

    spec2 = ((0, 0, 0, BIG, None), (1, 1, 1, BIG, None),
             (2, 2, 2, BIG, None))
    agg2_ci, agg2_wr, agg2_rh = _make_seg_sum(spec2, 3, 3, 3)(
        x1_p, x1_a, x1_f, s_ci, s_wr, s_rh, d_ci, d_wr, d_rh, zeros_big)

    x2_p = _sage_matmul([agg2_ci, agg2_wr, agg2_rh],
                        [cnt_ci, cnt_wr, cnt_rh],
                        x1_p, w_cat(1, (0, 1, 6)), b_sum(1, (0, 1, 6)),
                        m=n_paper, relu=True, out_chunked=False,
                        root_chunked=True)

    n_out = lin_W.shape[0]
    lin_bias = jnp.broadcast_to(lin_b[None, :], (8, n_out))
    return _final_linear(x2_p, lin_W.T, lin_bias, m=n_paper, n_out=n_out)
